# Initial kernel scaffold; baseline (speedup 1.0000x reference)
import jax
import jax.numpy as jnp
from jax.experimental import pallas as pl


def kernel(_input, dependency_triples, W_self, b_self, W_dep, b_dep, W_rev, b_rev):
    raise NotImplementedError("write your pallas kernel here")


# speedup vs baseline: 1.4604x; 1.4604x over previous
import functools

import jax
import jax.numpy as jnp
from jax import lax
from jax.experimental import pallas as pl
from jax.experimental.pallas import tpu as pltpu
from jax.experimental.pallas import tpu_sc as plsc

NC = 2
NS = 16
NW = NC * NS
LANES = 16
CHI = 112


def _pad_to(e, m):
    return ((e + m - 1) // m) * m


def _make_gather(N, D, EP):
    g_per_w = EP // NW
    n_ch = g_per_w // CHI
    mesh = plsc.VectorSubcoreMesh(
        core_axis_name="c", subcore_axis_name="s", num_cores=NC, num_subcores=NS
    )

    @functools.partial(
        pl.kernel,
        out_type=(
            jax.ShapeDtypeStruct((EP, D), jnp.float32),
            jax.ShapeDtypeStruct((EP, D), jnp.float32),
        ),
        mesh=mesh,
        scratch_types=[
            pltpu.VMEM((CHI,), jnp.int32),
            pltpu.VMEM((CHI, D), jnp.float32),
            pltpu.SemaphoreType.DMA,
        ],
        compiler_params=pltpu.CompilerParams(use_tc_tiling_on_sc=False),
    )
    def gather_k(x_hbm, gov_hbm, dep_hbm, xg_hbm, xd_hbm, idx_v, rows_v, sem):
        wid = lax.axis_index("s") * NC + lax.axis_index("c")
        base = wid * g_per_w

        def chunk(p, _):
            off = base + p * CHI
            pltpu.sync_copy(gov_hbm.at[pl.ds(off, CHI)], idx_v)
            pltpu.async_copy(x_hbm.at[idx_v], rows_v, sem).wait()
            pltpu.sync_copy(rows_v, xg_hbm.at[pl.ds(off, CHI)])
            pltpu.sync_copy(dep_hbm.at[pl.ds(off, CHI)], idx_v)
            pltpu.async_copy(x_hbm.at[idx_v], rows_v, sem).wait()
            pltpu.sync_copy(rows_v, xd_hbm.at[pl.ds(off, CHI)])
            return 0

        lax.fori_loop(0, n_ch, chunk, 0)

    return gather_k


def _make_dense(N, D, L, EP, BLK):
    def tc_body(x_ref, xg_ref, xd_ref, lab_ref, ws, bs, wd, bd, wr, br,
                o0_ref, mf_ref, mr_ref):
        x = x_ref[...]
        o0_ref[...] = (
            jnp.dot(x, ws[...], preferred_element_type=jnp.float32) + bs[...]
        )
        lab = lab_ref[...] % L
        xg = xg_ref[...]
        xd = xd_ref[...]
        mf = jnp.zeros((BLK, D), jnp.float32)
        mr = jnp.zeros((BLK, D), jnp.float32)
        for l in range(L):
            m = (lab == l).astype(jnp.float32)
            mf = mf + m * (
                jnp.dot(xg, wd[l], preferred_element_type=jnp.float32) + bd[l]
            )
            mr = mr + m * (
                jnp.dot(xd, wr[l], preferred_element_type=jnp.float32) + br[l]
            )
        mf_ref[...] = mf
        mr_ref[...] = mr

    row = lambda i: (i, 0)
    whole2 = lambda i: (0, 0)
    whole3 = lambda i: (0, 0, 0)
    return pl.pallas_call(
        tc_body,
        grid=(EP // BLK,),
        in_specs=[
            pl.BlockSpec((BLK, D), row),
            pl.BlockSpec((BLK, D), row),
            pl.BlockSpec((BLK, D), row),
            pl.BlockSpec((BLK, 1), row),
            pl.BlockSpec((D, D), whole2),
            pl.BlockSpec((1, D), whole2),
            pl.BlockSpec((L, D, D), whole3),
            pl.BlockSpec((L, 1, D), whole3),
            pl.BlockSpec((L, D, D), whole3),
            pl.BlockSpec((L, 1, D), whole3),
        ],
        out_specs=[
            pl.BlockSpec((BLK, D), row),
            pl.BlockSpec((BLK, D), row),
            pl.BlockSpec((BLK, D), row),
        ],
        out_shape=[
            jax.ShapeDtypeStruct((EP, D), jnp.float32),
            jax.ShapeDtypeStruct((EP, D), jnp.float32),
            jax.ShapeDtypeStruct((EP, D), jnp.float32),
        ],
        compiler_params=pltpu.CompilerParams(
            dimension_semantics=("parallel",)
        ),
    )


def _make_scatter(N, D, EP):
    HALF = EP // 2
    BUF = HALF + LANES
    D2 = D // 2
    s_per_t = EP // NS
    SLAB = s_per_t // 2
    n_pc = SLAB // CHI
    wb = HALF // NS
    mesh = plsc.VectorSubcoreMesh(
        core_axis_name="c", subcore_axis_name="s", num_cores=NC, num_subcores=NS
    )

    @functools.partial(
        pl.kernel,
        out_type=jax.ShapeDtypeStruct((EP, D), jnp.float32),
        mesh=mesh,
        scratch_types=[
            pltpu.VMEM_SHARED((BUF, D2), jnp.float32),
            pltpu.VMEM((SLAB,), jnp.int32),
            pltpu.VMEM((CHI,), jnp.int32),
            pltpu.VMEM((SLAB, D2), jnp.float32),
        ],
        compiler_params=pltpu.CompilerParams(use_tc_tiling_on_sc=False),
    )
    def scatter_k(o0_hbm, mf_hbm, mr_hbm, dep_hbm, gov_hbm, out_hbm,
                  acc_sh, di_v, idx_v, m_v):
        c = lax.axis_index("c")
        s = lax.axis_index("s")
        lo = c * HALF
        r0 = s * wb
        base = s * s_per_t
        for cp in range(D // D2):
            cols = pl.ds(cp * D2, D2)
            pltpu.sync_copy(
                o0_hbm.at[pl.ds(lo + r0, wb), cols], acc_sh.at[pl.ds(r0, wb)]
            )
            plsc.subcore_barrier()

            for k in range(s_per_t // SLAB):
                off = base + k * SLAB
                for ind_hbm, msg_hbm in ((dep_hbm, mf_hbm), (gov_hbm, mr_hbm)):
                    pltpu.sync_copy(ind_hbm.at[pl.ds(off, SLAB)], di_v)
                    pltpu.sync_copy(msg_hbm.at[pl.ds(off, SLAB), cols], m_v)

                    def piece(p, _):
                        for j in range(CHI // LANES):
                            v = di_v[pl.ds(p * CHI + j * LANES, LANES)]
                            inr = (v >= lo) & (v < lo + HALF)
                            idx_v[pl.ds(j * LANES, LANES)] = jnp.where(
                                inr, v - lo, HALF
                            )
                        pltpu.sync_copy(
                            m_v.at[pl.ds(p * CHI, CHI)], acc_sh.at[idx_v],
                            add=True,
                        )
                        return 0

                    lax.fori_loop(0, n_pc, piece, 0)
            plsc.subcore_barrier()

            pltpu.sync_copy(acc_sh.at[pl.ds(r0, wb)], m_v.at[pl.ds(0, wb)])

            def relu_row(i, _):
                for j in range(D2 // LANES):
                    t = m_v[i, pl.ds(j * LANES, LANES)]
                    m_v[i, pl.ds(j * LANES, LANES)] = jnp.maximum(t, 0.0)
                return 0

            lax.fori_loop(0, wb, relu_row, 0)
            pltpu.sync_copy(
                m_v.at[pl.ds(0, wb)], out_hbm.at[pl.ds(lo + r0, wb), cols]
            )

    return scatter_k


def kernel(_input, dependency_triples, W_self, b_self, W_dep, b_dep, W_rev, b_rev):
    N, D = _input.shape
    E = dependency_triples.shape[0]
    L = W_dep.shape[0]
    BLK = 1024
    EP = _pad_to(max(E, N), 7168)

    dep = dependency_triples[:, 0]
    lab = dependency_triples[:, 1]
    gov = dependency_triples[:, 2]
    padn = EP - E
    zpad = jnp.zeros((padn,), jnp.int32)
    npad = jnp.full((padn,), N, jnp.int32)
    gov_g = jnp.concatenate([gov, zpad])
    dep_g = jnp.concatenate([dep, zpad])
    dep_s = jnp.concatenate([dep, npad])
    gov_s = jnp.concatenate([gov, npad])
    lab2 = jnp.concatenate([lab, zpad]).reshape(EP, 1)

    xg, xd = _make_gather(N, D, EP)(_input, gov_g, dep_g)
    out0, mf, mr = _make_dense(N, D, L, EP, BLK)(
        _input, xg, xd, lab2,
        W_self, b_self.reshape(1, D),
        W_dep, b_dep.reshape(L, 1, D),
        W_rev, b_rev.reshape(L, 1, D),
    )
    outp = _make_scatter(N, D, EP)(out0, mf, mr, dep_s, gov_s)
    return outp[:N]


# speedup vs baseline: 1.9214x; 1.3156x over previous
import functools

import jax
import jax.numpy as jnp
from jax import lax
from jax.experimental import pallas as pl
from jax.experimental.pallas import tpu as pltpu
from jax.experimental.pallas import tpu_sc as plsc

NC = 2
NS = 16
NW = NC * NS
LANES = 16
CHI = 112


def _pad_to(e, m):
    return ((e + m - 1) // m) * m


def _make_gather(N, D, EP):
    g_per_w = EP // NW
    n_ch = g_per_w // CHI
    mesh = plsc.VectorSubcoreMesh(
        core_axis_name="c", subcore_axis_name="s", num_cores=NC, num_subcores=NS
    )

    @functools.partial(
        pl.kernel,
        out_type=(
            jax.ShapeDtypeStruct((EP, D), jnp.float32),
            jax.ShapeDtypeStruct((EP, D), jnp.float32),
        ),
        mesh=mesh,
        scratch_types=[
            pltpu.VMEM((CHI,), jnp.int32),
            pltpu.VMEM((CHI, D), jnp.float32),
            pltpu.SemaphoreType.DMA,
        ],
        compiler_params=pltpu.CompilerParams(use_tc_tiling_on_sc=False),
    )
    def gather_k(x_hbm, gov_hbm, dep_hbm, xg_hbm, xd_hbm, idx_v, rows_v, sem):
        wid = lax.axis_index("s") * NC + lax.axis_index("c")
        base = wid * g_per_w

        def chunk(p, _):
            off = base + p * CHI
            pltpu.sync_copy(gov_hbm.at[pl.ds(off, CHI)], idx_v)
            pltpu.async_copy(x_hbm.at[idx_v], rows_v, sem).wait()
            pltpu.sync_copy(rows_v, xg_hbm.at[pl.ds(off, CHI)])
            pltpu.sync_copy(dep_hbm.at[pl.ds(off, CHI)], idx_v)
            pltpu.async_copy(x_hbm.at[idx_v], rows_v, sem).wait()
            pltpu.sync_copy(rows_v, xd_hbm.at[pl.ds(off, CHI)])
            return 0

        lax.fori_loop(0, n_ch, chunk, 0)

    return gather_k


def _make_dense(N, D, L, EP, BLK):
    B2 = BLK // 2
    TD = 2 * D
    BN = BLK * D

    def tc_body(x_ref, xg_ref, xd_ref, lq_ref, ws2, bs2, wd2, bd2, wr2, br2,
                o0_ref, mf_ref, mr_ref):
        px = x_ref[...].reshape(B2, TD)
        o0 = jnp.dot(px, ws2[...], preferred_element_type=jnp.float32) + bs2[...]
        o0_ref[...] = o0.reshape(BN)

        Lq = lq_ref[...].reshape(B2, TD) % L
        masks = [(Lq == l).astype(jnp.float32) for l in range(L)]

        def stacked(p, wstack, bmat):
            bias = masks[0] * bmat[0]
            for l in range(1, L):
                bias = bias + masks[l] * bmat[l]
            xcat = jnp.concatenate([m * p for m in masks], axis=1)
            return (
                jnp.dot(xcat, wstack[...], preferred_element_type=jnp.float32)
                + bias
            )

        pg = xg_ref[...].reshape(B2, TD)
        mf_ref[...] = stacked(pg, wd2, bd2[...]).reshape(BN)
        pd = xd_ref[...].reshape(B2, TD)
        mr_ref[...] = stacked(pd, wr2, br2[...]).reshape(BN)

    flat = lambda i: (i,)
    whole2 = lambda i: (0, 0)
    whole3 = lambda i: (0, 0, 0)
    return pl.pallas_call(
        tc_body,
        grid=(EP // BLK,),
        in_specs=[
            pl.BlockSpec((BN,), flat),
            pl.BlockSpec((BN,), flat),
            pl.BlockSpec((BN,), flat),
            pl.BlockSpec((BN,), flat),
            pl.BlockSpec((TD, TD), whole2),
            pl.BlockSpec((1, TD), whole2),
            pl.BlockSpec((L * TD, TD), whole2),
            pl.BlockSpec((L, 1, TD), whole3),
            pl.BlockSpec((L * TD, TD), whole2),
            pl.BlockSpec((L, 1, TD), whole3),
        ],
        out_specs=[
            pl.BlockSpec((BN,), flat),
            pl.BlockSpec((BN,), flat),
            pl.BlockSpec((BN,), flat),
        ],
        out_shape=[
            jax.ShapeDtypeStruct((EP * D,), jnp.float32),
            jax.ShapeDtypeStruct((EP * D,), jnp.float32),
            jax.ShapeDtypeStruct((EP * D,), jnp.float32),
        ],
        compiler_params=pltpu.CompilerParams(
            dimension_semantics=("parallel",)
        ),
    )


def _make_scatter(N, D, EP):
    HALF = EP // 2
    BUF = HALF + LANES
    D2 = D // 2
    s_per_t = EP // NS
    SLAB = s_per_t // 2
    n_pc = SLAB // CHI
    wb = HALF // NS
    mesh = plsc.VectorSubcoreMesh(
        core_axis_name="c", subcore_axis_name="s", num_cores=NC, num_subcores=NS
    )

    @functools.partial(
        pl.kernel,
        out_type=jax.ShapeDtypeStruct((EP, D), jnp.float32),
        mesh=mesh,
        scratch_types=[
            pltpu.VMEM_SHARED((BUF, D2), jnp.float32),
            pltpu.VMEM((SLAB,), jnp.int32),
            pltpu.VMEM((CHI,), jnp.int32),
            pltpu.VMEM((SLAB, D2), jnp.float32),
        ],
        compiler_params=pltpu.CompilerParams(use_tc_tiling_on_sc=False),
    )
    def scatter_k(o0_hbm, mf_hbm, mr_hbm, dep_hbm, gov_hbm, out_hbm,
                  acc_sh, di_v, idx_v, m_v):
        c = lax.axis_index("c")
        s = lax.axis_index("s")
        lo = c * HALF
        r0 = s * wb
        base = s * s_per_t
        for cp in range(D // D2):
            cols = pl.ds(cp * D2, D2)
            pltpu.sync_copy(
                o0_hbm.at[pl.ds(lo + r0, wb), cols], acc_sh.at[pl.ds(r0, wb)]
            )
            plsc.subcore_barrier()

            for k in range(s_per_t // SLAB):
                off = base + k * SLAB
                for ind_hbm, msg_hbm in ((dep_hbm, mf_hbm), (gov_hbm, mr_hbm)):
                    pltpu.sync_copy(ind_hbm.at[pl.ds(off, SLAB)], di_v)
                    pltpu.sync_copy(msg_hbm.at[pl.ds(off, SLAB), cols], m_v)

                    def piece(p, _):
                        for j in range(CHI // LANES):
                            v = di_v[pl.ds(p * CHI + j * LANES, LANES)]
                            inr = (v >= lo) & (v < lo + HALF)
                            idx_v[pl.ds(j * LANES, LANES)] = jnp.where(
                                inr, v - lo, HALF
                            )
                        pltpu.sync_copy(
                            m_v.at[pl.ds(p * CHI, CHI)], acc_sh.at[idx_v],
                            add=True,
                        )
                        return 0

                    lax.fori_loop(0, n_pc, piece, 0)
            plsc.subcore_barrier()

            pltpu.sync_copy(
                acc_sh.at[pl.ds(r0, wb)], out_hbm.at[pl.ds(lo + r0, wb), cols]
            )
            plsc.subcore_barrier()

    return scatter_k


def _make_relu_unpack(N, D, EP, BLK):
    B2 = BLK // 2
    BN = BLK * D

    def body(in_ref, o_ref):
        p = in_ref[...].reshape(B2, 2 * D)
        a = p[:, :D].reshape(B2, 1, D)
        b = p[:, D:].reshape(B2, 1, D)
        y = jnp.concatenate([a, b], axis=1).reshape(BLK, D)
        o_ref[...] = jnp.maximum(y, 0.0)

    return pl.pallas_call(
        body,
        grid=(EP // BLK,),
        in_specs=[pl.BlockSpec((BN,), lambda i: (i,))],
        out_specs=pl.BlockSpec((BLK, D), lambda i: (i, 0)),
        out_shape=jax.ShapeDtypeStruct((N, D), jnp.float32),
        compiler_params=pltpu.CompilerParams(dimension_semantics=("parallel",)),
    )


def _blockdiag2(w):
    z = jnp.zeros_like(w)
    top = jnp.concatenate([w, z], axis=-1)
    bot = jnp.concatenate([z, w], axis=-1)
    return jnp.concatenate([top, bot], axis=-2)


def kernel(_input, dependency_triples, W_self, b_self, W_dep, b_dep, W_rev, b_rev):
    N, D = _input.shape
    E = dependency_triples.shape[0]
    L = W_dep.shape[0]
    BLK = 1024
    EP = _pad_to(max(E, N), 7168)
    TD = 2 * D

    dep = dependency_triples[:, 0]
    lab = dependency_triples[:, 1]
    gov = dependency_triples[:, 2]
    padn = EP - E
    zpad = jnp.zeros((padn,), jnp.int32)
    npad = jnp.full((padn,), N, jnp.int32)
    gov_g = jnp.concatenate([gov, zpad])
    dep_g = jnp.concatenate([dep, zpad])
    dep_s = jnp.concatenate([dep, npad])
    gov_s = jnp.concatenate([gov, npad])
    labq = jnp.repeat(jnp.concatenate([lab, zpad]), D)

    ws2 = _blockdiag2(W_self)
    wd2 = _blockdiag2(W_dep).reshape(L * TD, TD)
    wr2 = _blockdiag2(W_rev).reshape(L * TD, TD)
    bs2 = jnp.tile(b_self, 2).reshape(1, TD)
    bd2 = jnp.tile(b_dep, (1, 2)).reshape(L, 1, TD)
    br2 = jnp.tile(b_rev, (1, 2)).reshape(L, 1, TD)

    x1d = _input.reshape(N * D)
    xg, xd = _make_gather(N, D, EP)(x1d.reshape(N, D), gov_g, dep_g)

    o01, mf1, mr1 = _make_dense(N, D, L, EP, BLK)(
        x1d,
        xg.reshape(EP * D),
        xd.reshape(EP * D),
        labq,
        ws2, bs2, wd2, bd2, wr2, br2,
    )
    outp = _make_scatter(N, D, EP)(
        o01.reshape(EP, D),
        mf1.reshape(EP, D),
        mr1.reshape(EP, D),
        dep_s, gov_s,
    )
    return _make_relu_unpack(N, D, EP, BLK)(outp.reshape(EP * D))


# speedup vs baseline: 2.5121x; 1.3074x over previous
import functools

import jax
import jax.numpy as jnp
from jax import lax
from jax.experimental import pallas as pl
from jax.experimental.pallas import tpu as pltpu
from jax.experimental.pallas import tpu_sc as plsc

NC = 2
NS = 16
NW = NC * NS
LANES = 16
CHI = 112


def _pad_to(e, m):
    return ((e + m - 1) // m) * m


def _make_gather(N, D, EP):
    g_per_w = EP // NW
    n_ch = g_per_w // CHI
    mesh = plsc.VectorSubcoreMesh(
        core_axis_name="c", subcore_axis_name="s", num_cores=NC, num_subcores=NS
    )

    @functools.partial(
        pl.kernel,
        out_type=(
            jax.ShapeDtypeStruct((EP, D), jnp.float32),
            jax.ShapeDtypeStruct((EP, D), jnp.float32),
        ),
        mesh=mesh,
        scratch_types=[
            pltpu.VMEM((g_per_w,), jnp.int32),
            pltpu.VMEM((g_per_w, D), jnp.float32),
            pltpu.SemaphoreType.DMA,
        ],
        compiler_params=pltpu.CompilerParams(use_tc_tiling_on_sc=False),
    )
    def gather_k(x_hbm, gov_hbm, dep_hbm, xg_hbm, xd_hbm, idx_v, rows_v, sem):
        wid = lax.axis_index("s") * NC + lax.axis_index("c")
        base = wid * g_per_w

        for ind_hbm, out_hbm in ((gov_hbm, xg_hbm), (dep_hbm, xd_hbm)):
            pltpu.sync_copy(ind_hbm.at[pl.ds(base, g_per_w)], idx_v)
            descs = [
                pltpu.async_copy(
                    x_hbm.at[idx_v.at[pl.ds(p * CHI, CHI)]],
                    rows_v.at[pl.ds(p * CHI, CHI)],
                    sem,
                )
                for p in range(n_ch)
            ]
            for d in descs:
                d.wait()
            pltpu.sync_copy(rows_v, out_hbm.at[pl.ds(base, g_per_w)])

    return gather_k


def _make_dense(N, D, L, EP, BLK):
    B2 = BLK // 2
    TD = 2 * D
    BN = BLK * D

    def tc_body(x_ref, xg_ref, xd_ref, lq_ref, ws2, bs2, wd2, bd2, wr2, br2,
                o0_ref, mf_ref, mr_ref):
        px = x_ref[...].reshape(B2, TD)
        o0 = jnp.dot(px, ws2[...], preferred_element_type=jnp.float32) + bs2[...]
        o0_ref[...] = o0.reshape(BN)

        Lq = lq_ref[...].reshape(B2, TD) % L
        masks = [(Lq == l).astype(jnp.float32) for l in range(L)]

        def stacked(p, wstack, bmat):
            bias = masks[0] * bmat[0]
            for l in range(1, L):
                bias = bias + masks[l] * bmat[l]
            xcat = jnp.concatenate([m * p for m in masks], axis=1)
            return (
                jnp.dot(xcat, wstack[...], preferred_element_type=jnp.float32)
                + bias
            )

        pg = xg_ref[...].reshape(B2, TD)
        mf_ref[...] = stacked(pg, wd2, bd2[...]).reshape(BN)
        pd = xd_ref[...].reshape(B2, TD)
        mr_ref[...] = stacked(pd, wr2, br2[...]).reshape(BN)

    flat = lambda i: (i,)
    whole2 = lambda i: (0, 0)
    whole3 = lambda i: (0, 0, 0)
    return pl.pallas_call(
        tc_body,
        grid=(EP // BLK,),
        in_specs=[
            pl.BlockSpec((BN,), flat),
            pl.BlockSpec((BN,), flat),
            pl.BlockSpec((BN,), flat),
            pl.BlockSpec((BN,), flat),
            pl.BlockSpec((TD, TD), whole2),
            pl.BlockSpec((1, TD), whole2),
            pl.BlockSpec((L * TD, TD), whole2),
            pl.BlockSpec((L, 1, TD), whole3),
            pl.BlockSpec((L * TD, TD), whole2),
            pl.BlockSpec((L, 1, TD), whole3),
        ],
        out_specs=[
            pl.BlockSpec((BN,), flat),
            pl.BlockSpec((BN,), flat),
            pl.BlockSpec((BN,), flat),
        ],
        out_shape=[
            jax.ShapeDtypeStruct((EP * D,), jnp.float32),
            jax.ShapeDtypeStruct((EP * D,), jnp.float32),
            jax.ShapeDtypeStruct((EP * D,), jnp.float32),
        ],
        compiler_params=pltpu.CompilerParams(
            dimension_semantics=("parallel",)
        ),
    )


def _make_scatter(N, D, EP):
    HALF = EP // 2
    BUF = HALF + LANES
    D2 = D // 2
    s_per_t = EP // NS
    SLAB = s_per_t // 2
    n_pc = SLAB // CHI
    wb = HALF // NS
    mesh = plsc.VectorSubcoreMesh(
        core_axis_name="c", subcore_axis_name="s", num_cores=NC, num_subcores=NS
    )

    @functools.partial(
        pl.kernel,
        out_type=jax.ShapeDtypeStruct((EP, D), jnp.float32),
        mesh=mesh,
        scratch_types=[
            pltpu.VMEM_SHARED((BUF, D2), jnp.float32),
            pltpu.VMEM((SLAB,), jnp.int32),
            pltpu.VMEM((n_pc, CHI), jnp.int32),
            pltpu.VMEM((SLAB, D2), jnp.float32),
            pltpu.SemaphoreType.DMA,
        ],
        compiler_params=pltpu.CompilerParams(use_tc_tiling_on_sc=False),
    )
    def scatter_k(o0_hbm, mf_hbm, mr_hbm, dep_hbm, gov_hbm, out_hbm,
                  acc_sh, di_v, idx2_v, m_v, sem):
        c = lax.axis_index("c")
        s = lax.axis_index("s")
        lo = c * HALF
        r0 = s * wb
        base = s * s_per_t
        lane = lax.iota(jnp.int32, LANES)
        for cp in range(D // D2):
            cols = pl.ds(cp * D2, D2)
            pltpu.sync_copy(
                o0_hbm.at[pl.ds(lo + r0, wb), cols], acc_sh.at[pl.ds(r0, wb)]
            )
            plsc.subcore_barrier()

            for k in range(s_per_t // SLAB):
                off = base + k * SLAB
                for ind_hbm, msg_hbm in ((dep_hbm, mf_hbm), (gov_hbm, mr_hbm)):
                    pltpu.sync_copy(ind_hbm.at[pl.ds(off, SLAB)], di_v)
                    pltpu.sync_copy(msg_hbm.at[pl.ds(off, SLAB), cols], m_v)

                    def piece(p, _):
                        for j in range(CHI // LANES):
                            v = di_v[pl.ds(p * CHI + j * LANES, LANES)]
                            inr = (v >= lo) & (v < lo + HALF)
                            idx2_v[p, pl.ds(j * LANES, LANES)] = jnp.where(
                                inr, v - lo, HALF + lane
                            )
                        return 0

                    lax.fori_loop(0, n_pc, piece, 0)
                    descs = [
                        pltpu.async_copy(
                            m_v.at[pl.ds(p * CHI, CHI)],
                            acc_sh.at[idx2_v.at[p]],
                            sem,
                            add=True,
                        )
                        for p in range(n_pc)
                    ]
                    for dsc in descs:
                        dsc.wait()
            plsc.subcore_barrier()

            pltpu.sync_copy(
                acc_sh.at[pl.ds(r0, wb)], out_hbm.at[pl.ds(lo + r0, wb), cols]
            )
            plsc.subcore_barrier()

    return scatter_k


def _make_relu_unpack(N, D, EP, BLK):
    B2 = BLK // 2
    BN = BLK * D

    def body(in_ref, o_ref):
        p = in_ref[...].reshape(B2, 2 * D)
        a = p[:, :D].reshape(B2, 1, D)
        b = p[:, D:].reshape(B2, 1, D)
        y = jnp.concatenate([a, b], axis=1).reshape(BLK, D)
        o_ref[...] = jnp.maximum(y, 0.0)

    return pl.pallas_call(
        body,
        grid=(EP // BLK,),
        in_specs=[pl.BlockSpec((BN,), lambda i: (i,))],
        out_specs=pl.BlockSpec((BLK, D), lambda i: (i, 0)),
        out_shape=jax.ShapeDtypeStruct((N, D), jnp.float32),
        compiler_params=pltpu.CompilerParams(dimension_semantics=("parallel",)),
    )


def _blockdiag2(w):
    z = jnp.zeros_like(w)
    top = jnp.concatenate([w, z], axis=-1)
    bot = jnp.concatenate([z, w], axis=-1)
    return jnp.concatenate([top, bot], axis=-2)


def kernel(_input, dependency_triples, W_self, b_self, W_dep, b_dep, W_rev, b_rev):
    N, D = _input.shape
    E = dependency_triples.shape[0]
    L = W_dep.shape[0]
    BLK = 1024
    EP = _pad_to(max(E, N), 7168)
    TD = 2 * D

    dep = dependency_triples[:, 0]
    lab = dependency_triples[:, 1]
    gov = dependency_triples[:, 2]
    padn = EP - E
    zpad = jnp.zeros((padn,), jnp.int32)
    npad = jnp.full((padn,), N, jnp.int32)
    gov_g = jnp.concatenate([gov, zpad])
    dep_g = jnp.concatenate([dep, zpad])
    dep_s = jnp.concatenate([dep, npad])
    gov_s = jnp.concatenate([gov, npad])
    labq = jnp.repeat(jnp.concatenate([lab, zpad]), D)

    ws2 = _blockdiag2(W_self)
    wd2 = _blockdiag2(W_dep).reshape(L * TD, TD)
    wr2 = _blockdiag2(W_rev).reshape(L * TD, TD)
    bs2 = jnp.tile(b_self, 2).reshape(1, TD)
    bd2 = jnp.tile(b_dep, (1, 2)).reshape(L, 1, TD)
    br2 = jnp.tile(b_rev, (1, 2)).reshape(L, 1, TD)

    x1d = _input.reshape(N * D)
    xg, xd = _make_gather(N, D, EP)(x1d.reshape(N, D), gov_g, dep_g)

    o01, mf1, mr1 = _make_dense(N, D, L, EP, BLK)(
        x1d,
        xg.reshape(EP * D),
        xd.reshape(EP * D),
        labq,
        ws2, bs2, wd2, bd2, wr2, br2,
    )
    outp = _make_scatter(N, D, EP)(
        o01.reshape(EP, D),
        mf1.reshape(EP, D),
        mr1.reshape(EP, D),
        dep_s, gov_s,
    )
    return _make_relu_unpack(N, D, EP, BLK)(outp.reshape(EP * D))


# speedup vs baseline: 2.6807x; 1.0671x over previous
import functools

import jax
import jax.numpy as jnp
from jax import lax
from jax.experimental import pallas as pl
from jax.experimental.pallas import tpu as pltpu
from jax.experimental.pallas import tpu_sc as plsc

NC = 2
NS = 16
NW = NC * NS
LANES = 16
CHI = 112


def _pad_to(e, m):
    return ((e + m - 1) // m) * m


def _make_gather(N, D, EP):
    g_per_w = EP // NW
    n_ch = g_per_w // CHI
    mesh = plsc.VectorSubcoreMesh(
        core_axis_name="c", subcore_axis_name="s", num_cores=NC, num_subcores=NS
    )

    @functools.partial(
        pl.kernel,
        out_type=(
            jax.ShapeDtypeStruct((EP, D), jnp.float32),
            jax.ShapeDtypeStruct((EP, D), jnp.float32),
        ),
        mesh=mesh,
        scratch_types=[
            pltpu.VMEM((g_per_w,), jnp.int32),
            pltpu.VMEM((g_per_w, D), jnp.float32),
            pltpu.SemaphoreType.DMA,
        ],
        compiler_params=pltpu.CompilerParams(use_tc_tiling_on_sc=False),
    )
    def gather_k(x_hbm, gov_hbm, dep_hbm, xg_hbm, xd_hbm, idx_v, rows_v, sem):
        wid = lax.axis_index("s") * NC + lax.axis_index("c")
        base = wid * g_per_w

        for ind_hbm, out_hbm in ((gov_hbm, xg_hbm), (dep_hbm, xd_hbm)):
            pltpu.sync_copy(ind_hbm.at[pl.ds(base, g_per_w)], idx_v)
            descs = [
                pltpu.async_copy(
                    x_hbm.at[idx_v.at[pl.ds(p * CHI, CHI)]],
                    rows_v.at[pl.ds(p * CHI, CHI)],
                    sem,
                )
                for p in range(n_ch)
            ]
            for d in descs:
                d.wait()
            pltpu.sync_copy(rows_v, out_hbm.at[pl.ds(base, g_per_w)])

    return gather_k


def _make_dense(N, D, L, EP, BLK):
    B2 = BLK // 2
    TD = 2 * D
    BN = BLK * D

    def tc_body(x_ref, xg_ref, xd_ref, lq_ref, ws2, bs2, wd2, bd2, wr2, br2,
                o0_ref, mf_ref, mr_ref):
        px = x_ref[...].reshape(B2, TD)
        o0 = jnp.dot(px, ws2[...], preferred_element_type=jnp.float32) + bs2[...]
        o0_ref[...] = o0.reshape(BN)

        Lq = lq_ref[...].reshape(B2, TD).astype(jnp.int32)
        masks = [(Lq == l).astype(jnp.float32) for l in range(L)]

        def stacked(p, wstack, bmat):
            bias = masks[0] * bmat[0]
            for l in range(1, L):
                bias = bias + masks[l] * bmat[l]
            xcat = jnp.concatenate([m * p for m in masks], axis=1)
            return (
                jnp.dot(xcat, wstack[...], preferred_element_type=jnp.float32)
                + bias
            )

        pg = xg_ref[...].reshape(B2, TD)
        mf_ref[...] = stacked(pg, wd2, bd2[...]).reshape(BN)
        pd = xd_ref[...].reshape(B2, TD)
        mr_ref[...] = stacked(pd, wr2, br2[...]).reshape(BN)

    flat = lambda i: (i,)
    whole2 = lambda i: (0, 0)
    whole3 = lambda i: (0, 0, 0)
    return pl.pallas_call(
        tc_body,
        grid=(EP // BLK,),
        in_specs=[
            pl.BlockSpec((BN,), flat),
            pl.BlockSpec((BN,), flat),
            pl.BlockSpec((BN,), flat),
            pl.BlockSpec((BN,), flat),
            pl.BlockSpec((TD, TD), whole2),
            pl.BlockSpec((1, TD), whole2),
            pl.BlockSpec((L * TD, TD), whole2),
            pl.BlockSpec((L, 1, TD), whole3),
            pl.BlockSpec((L * TD, TD), whole2),
            pl.BlockSpec((L, 1, TD), whole3),
        ],
        out_specs=[
            pl.BlockSpec((BN,), flat),
            pl.BlockSpec((BN,), flat),
            pl.BlockSpec((BN,), flat),
        ],
        out_shape=[
            jax.ShapeDtypeStruct((EP * D,), jnp.float32),
            jax.ShapeDtypeStruct((EP * D,), jnp.float32),
            jax.ShapeDtypeStruct((EP * D,), jnp.float32),
        ],
        compiler_params=pltpu.CompilerParams(
            dimension_semantics=("parallel",)
        ),
    )


def _make_scatter(N, D, EP):
    HALF = EP // 2
    BUF = HALF + LANES
    D2 = D // 2
    s_per_t = EP // NS
    SLAB = s_per_t // 4
    n_pc = SLAB // CHI
    wb = HALF // NS
    mesh = plsc.VectorSubcoreMesh(
        core_axis_name="c", subcore_axis_name="s", num_cores=NC, num_subcores=NS
    )

    @functools.partial(
        pl.kernel,
        out_type=jax.ShapeDtypeStruct((EP, D), jnp.float32),
        mesh=mesh,
        scratch_types=[
            pltpu.VMEM_SHARED((BUF, D2), jnp.float32),
            [pltpu.VMEM((SLAB,), jnp.int32)] * 2,
            [pltpu.VMEM((n_pc, CHI), jnp.int32)] * 2,
            [pltpu.VMEM((SLAB, D2), jnp.float32)] * 2,
            pltpu.SemaphoreType.DMA,
            pltpu.SemaphoreType.DMA,
        ],
        compiler_params=pltpu.CompilerParams(use_tc_tiling_on_sc=False),
    )
    def scatter_k(o0_hbm, mf_hbm, mr_hbm, dep_hbm, gov_hbm, out_hbm,
                  acc_sh, di_v, idx2_v, m_v, sem, sem2):
        c = lax.axis_index("c")
        s = lax.axis_index("s")
        lo = c * HALF
        r0 = s * wb
        base = s * s_per_t
        lane = lax.iota(jnp.int32, LANES)

        def compute_idx(b):
            def piece(p, _):
                for j in range(CHI // LANES):
                    v = di_v[b][pl.ds(p * CHI + j * LANES, LANES)]
                    inr = (v >= lo) & (v < lo + HALF)
                    idx2_v[b][p, pl.ds(j * LANES, LANES)] = jnp.where(
                        inr, v - lo, HALF + lane
                    )
                return 0

            lax.fori_loop(0, n_pc, piece, 0)

        for cp in range(D // D2):
            cols = pl.ds(cp * D2, D2)
            pltpu.sync_copy(
                o0_hbm.at[pl.ds(lo + r0, wb), cols], acc_sh.at[pl.ds(r0, wb)]
            )
            plsc.subcore_barrier()

            units = [
                (base + k * SLAB, ind_hbm, msg_hbm)
                for k in range(s_per_t // SLAB)
                for ind_hbm, msg_hbm in ((dep_hbm, mf_hbm), (gov_hbm, mr_hbm))
            ]
            off0, ind0, msg0 = units[0]
            pltpu.sync_copy(ind0.at[pl.ds(off0, SLAB)], di_v[0])
            pltpu.sync_copy(msg0.at[pl.ds(off0, SLAB), cols], m_v[0])
            compute_idx(0)
            b = 0
            for u, (off, ind_hbm, msg_hbm) in enumerate(units):
                descs = [
                    pltpu.async_copy(
                        m_v[b].at[pl.ds(p * CHI, CHI)],
                        acc_sh.at[idx2_v[b].at[p]],
                        sem,
                        add=True,
                    )
                    for p in range(n_pc)
                ]
                if u + 1 < len(units):
                    noff, nind, nmsg = units[u + 1]
                    mload = pltpu.async_copy(
                        nmsg.at[pl.ds(noff, SLAB), cols], m_v[1 - b], sem2
                    )
                    pltpu.sync_copy(nind.at[pl.ds(noff, SLAB)], di_v[1 - b])
                    compute_idx(1 - b)
                    for dsc in descs:
                        dsc.wait()
                    mload.wait()
                else:
                    for dsc in descs:
                        dsc.wait()
                b = 1 - b
            plsc.subcore_barrier()

            pltpu.sync_copy(
                acc_sh.at[pl.ds(r0, wb)], out_hbm.at[pl.ds(lo + r0, wb), cols]
            )
            plsc.subcore_barrier()

    return scatter_k


def _make_relu_unpack(N, D, EP, BLK):
    B2 = BLK // 2
    BN = BLK * D

    def body(in_ref, o_ref):
        p = in_ref[...].reshape(B2, 2 * D)
        a = p[:, :D].reshape(B2, 1, D)
        b = p[:, D:].reshape(B2, 1, D)
        y = jnp.concatenate([a, b], axis=1).reshape(BLK, D)
        o_ref[...] = jnp.maximum(y, 0.0)

    return pl.pallas_call(
        body,
        grid=(EP // BLK,),
        in_specs=[pl.BlockSpec((BN,), lambda i: (i,))],
        out_specs=pl.BlockSpec((BLK, D), lambda i: (i, 0)),
        out_shape=jax.ShapeDtypeStruct((N, D), jnp.float32),
        compiler_params=pltpu.CompilerParams(dimension_semantics=("parallel",)),
    )


def _blockdiag2(w):
    z = jnp.zeros_like(w)
    top = jnp.concatenate([w, z], axis=-1)
    bot = jnp.concatenate([z, w], axis=-1)
    return jnp.concatenate([top, bot], axis=-2)


def kernel(_input, dependency_triples, W_self, b_self, W_dep, b_dep, W_rev, b_rev):
    N, D = _input.shape
    E = dependency_triples.shape[0]
    L = W_dep.shape[0]
    BLK = 1024
    EP = _pad_to(max(E, N), 7168)
    TD = 2 * D

    dep = dependency_triples[:, 0]
    lab = dependency_triples[:, 1]
    gov = dependency_triples[:, 2]
    padn = EP - E
    zpad = jnp.zeros((padn,), jnp.int32)
    npad = jnp.full((padn,), N, jnp.int32)
    gov_g = jnp.concatenate([gov, zpad])
    dep_g = jnp.concatenate([dep, zpad])
    dep_s = jnp.concatenate([dep, npad])
    gov_s = jnp.concatenate([gov, npad])
    labq = jnp.repeat(jnp.concatenate([lab % L, zpad]).astype(jnp.int8), D)

    ws2 = _blockdiag2(W_self)
    wd2 = _blockdiag2(W_dep).reshape(L * TD, TD)
    wr2 = _blockdiag2(W_rev).reshape(L * TD, TD)
    bs2 = jnp.tile(b_self, 2).reshape(1, TD)
    bd2 = jnp.tile(b_dep, (1, 2)).reshape(L, 1, TD)
    br2 = jnp.tile(b_rev, (1, 2)).reshape(L, 1, TD)

    x1d = _input.reshape(N * D)
    xg, xd = _make_gather(N, D, EP)(x1d.reshape(N, D), gov_g, dep_g)

    o01, mf1, mr1 = _make_dense(N, D, L, EP, BLK)(
        x1d,
        xg.reshape(EP * D),
        xd.reshape(EP * D),
        labq,
        ws2, bs2, wd2, bd2, wr2, br2,
    )
    outp = _make_scatter(N, D, EP)(
        o01.reshape(EP, D),
        mf1.reshape(EP, D),
        mr1.reshape(EP, D),
        dep_s, gov_s,
    )
    return _make_relu_unpack(N, D, EP, BLK)(outp.reshape(EP * D))
